# baseline (speedup 1.0000x reference)
import functools

import jax
import jax.numpy as jnp
from jax import lax
from jax.experimental import pallas as pl
from jax.experimental.pallas import tpu as pltpu
from jax.experimental.pallas import tpu_sc as plsc

N = 10000
E = 320000
D = 128
K = 128
NW = 32
ECHUNKS = 80
EPT = ECHUNKS * K
EPAD = NW * EPT
EROWS = E // K
TRASH = N
ACC_ROWS = 10112
SROWS = ACC_ROWS // 16
RB = 1000
GRID = N // RB

_mesh = plsc.VectorSubcoreMesh(core_axis_name="c", subcore_axis_name="s")


@functools.partial(
    pl.kernel,
    out_type=jax.ShapeDtypeStruct((2 * NW * N,), jnp.float32),
    mesh=_mesh,
    scratch_types=[
        pltpu.VMEM((EPT,), jnp.int32),
        pltpu.VMEM((EPT,), jnp.int32),
        pltpu.VMEM((N,), jnp.float32),
        pltpu.VMEM((N,), jnp.float32),
    ],
    compiler_params=pltpu.CompilerParams(needs_layout_passes=False),
)
def _deg_kernel(edge_hbm, out_hbm, isrc_v, idst_v, dsrc_v, ddst_v):
    w = lax.axis_index("c") * 16 + lax.axis_index("s")
    pltpu.sync_copy(edge_hbm.at[pl.ds(w * EPT, EPT)], isrc_v)
    pltpu.sync_copy(edge_hbm.at[pl.ds(EPAD + w * EPT, EPT)], idst_v)

    zero16 = jnp.zeros((16,), jnp.float32)

    def zbody(i, _):
        dsrc_v[pl.ds(i * 16, 16)] = zero16
        ddst_v[pl.ds(i * 16, 16)] = zero16
        return 0

    lax.fori_loop(0, N // 16, zbody, 0)

    ones16 = jnp.ones((16,), jnp.float32)

    def body(k, _):
        @pl.when(w * ECHUNKS + k < EROWS)
        def _():
            for j in range(K // 16):
                s_idx = isrc_v[pl.ds(k * K + j * 16, 16)]
                d_idx = idst_v[pl.ds(k * K + j * 16, 16)]
                plsc.addupdate_scatter(dsrc_v, [s_idx], ones16)
                plsc.addupdate_scatter(ddst_v, [d_idx], ones16)
        return 0

    lax.fori_loop(0, ECHUNKS, body, 0)

    pltpu.sync_copy(dsrc_v, out_hbm.at[pl.ds(w * N, N)])
    pltpu.sync_copy(ddst_v, out_hbm.at[pl.ds((NW + w) * N, N)])


@functools.partial(
    pl.kernel,
    out_type=jax.ShapeDtypeStruct((2, ACC_ROWS, D), jnp.float32),
    mesh=_mesh,
    scratch_types=[
        pltpu.VMEM((3, 2, K), jnp.int32),
        pltpu.VMEM((K, D), jnp.float32),
        pltpu.VMEM((K, D), jnp.float32),
        pltpu.VMEM((K, D), jnp.float32),
        pltpu.VMEM_SHARED((ACC_ROWS, D), jnp.float32),
        pltpu.SemaphoreType.DMA,
        pltpu.SemaphoreType.DMA,
        pltpu.SemaphoreType.DMA,
        pltpu.SemaphoreType.DMA,
        pltpu.SemaphoreType.DMA,
        pltpu.SemaphoreType.DMA,
    ],
    compiler_params=pltpu.CompilerParams(needs_layout_passes=False),
)
def _agg_kernel(h_hbm, edge_hbm, out_hbm, idx_v, row0_v, row1_v, row2_v,
                acc_sh, g0, g1, g2, s0, s1, s2):
    c = lax.axis_index("c")
    s = lax.axis_index("s")
    w = c * 16 + s
    ebase = w * EPT

    rows = (row0_v, row1_v, row2_v)
    gsem = (g0, g1, g2)
    ssem = (s0, s1, s2)

    def load_idx(q, b):
        pltpu.sync_copy(edge_hbm.at[pl.ds(ebase + q * K, K)], idx_v.at[b, 0])
        pltpu.sync_copy(edge_hbm.at[pl.ds(EPAD + ebase + q * K, K)],
                        idx_v.at[b, 1])

    def start_gather(b):
        pltpu.async_copy(h_hbm.at[idx_v.at[b, 0]], rows[b], gsem[b])

    def wait_gather(b):
        pltpu.make_async_copy(h_hbm.at[idx_v.at[b, 0]], rows[b],
                              gsem[b]).wait()

    def start_scatter(b):
        pltpu.async_copy(rows[b], acc_sh.at[idx_v.at[b, 1]], ssem[b],
                         add=True)

    def wait_scatter(b):
        pltpu.make_async_copy(rows[b], acc_sh.at[idx_v.at[b, 1]],
                              ssem[b]).wait()

    load_idx(0, 0)
    start_gather(0)
    load_idx(1, 1)
    start_gather(1)

    zero16 = jnp.zeros((16,), jnp.float32)

    def zbody(i, _):
        for j in range(D // 16):
            row2_v[i, pl.ds(j * 16, 16)] = zero16
        return 0

    lax.fori_loop(0, K, zbody, 0)
    zbase = s * SROWS
    for t in range(SROWS // K):
        pltpu.sync_copy(row2_v, acc_sh.at[pl.ds(zbase + t * K, K)])
    pltpu.sync_copy(row2_v.at[pl.ds(0, SROWS % K)],
                    acc_sh.at[pl.ds(zbase + (SROWS // K) * K, SROWS % K)])
    plsc.subcore_barrier()

    def step(q, b, b2, first, last):
        wait_gather(b)
        start_scatter(b)
        if not last:
            if not first:
                wait_scatter(b2)
            load_idx(q + 2, b2)
            start_gather(b2)

    step(0, 0, 2, True, False)
    step(1, 1, 0, False, False)

    def body(m, _):
        q = 3 * m + 2
        step(q, 2, 1, False, False)
        step(q + 1, 0, 2, False, False)
        step(q + 2, 1, 0, False, False)
        return 0

    lax.fori_loop(0, 25, body, 0)
    step(77, 2, 1, False, False)
    step(78, 0, 2, False, True)
    step(79, 1, 0, False, True)
    wait_scatter(2)
    wait_scatter(0)
    wait_scatter(1)

    plsc.subcore_barrier()
    pltpu.sync_copy(acc_sh.at[pl.ds(s * SROWS, SROWS)],
                    out_hbm.at[c, pl.ds(s * SROWS, SROWS)])


def _rsq(deg_block):
    d = jnp.sum(deg_block, axis=1, keepdims=True)
    return lax.rsqrt(jnp.clip(d, 1.0, None))


def _mm(x, w_ref, b_ref):
    return lax.dot_general(
        x, w_ref[...], (((1,), (1,)), ((), ())),
        preferred_element_type=jnp.float32,
        precision=lax.Precision.HIGHEST,
    ) + b_ref[0:1, :]


def _tc_first_body(x_ref, w_ref, b_ref, degt_ref, o_ref):
    srs = _rsq(degt_ref[0])
    o_ref[...] = _mm(x_ref[...], w_ref, b_ref) * srs


def _tc_mid_body(agg_ref, degt_ref, x_ref, w_ref, b_ref, o_ref):
    irs = _rsq(degt_ref[1])
    srs = _rsq(degt_ref[0])
    p = agg_ref[0] + agg_ref[1]
    x1 = jnp.maximum(p * irs + x_ref[...], 0.0)
    o_ref[...] = _mm(x1, w_ref, b_ref) * srs


def _tc_final_body(agg_ref, degt_ref, o_ref):
    irs = _rsq(degt_ref[1])
    o_ref[...] = (agg_ref[0] + agg_ref[1]) * irs


_row_spec = pl.BlockSpec((RB, D), lambda i: (i, 0))
_w_spec = pl.BlockSpec((D, D), lambda i: (0, 0))
_b_spec = pl.BlockSpec((8, D), lambda i: (0, 0))
_degt_spec = pl.BlockSpec((2, RB, NW), lambda i: (0, i, 0))
_agg_spec = pl.BlockSpec((2, RB, D), lambda i: (0, i, 0))
_out_sds = jax.ShapeDtypeStruct((N, D), jnp.float32)

_tc_first = pl.pallas_call(
    _tc_first_body,
    grid=(GRID,),
    in_specs=[_row_spec, _w_spec, _b_spec, _degt_spec],
    out_specs=_row_spec,
    out_shape=_out_sds,
)

_tc_mid = pl.pallas_call(
    _tc_mid_body,
    grid=(GRID,),
    in_specs=[_agg_spec, _degt_spec, _row_spec, _w_spec, _b_spec],
    out_specs=_row_spec,
    out_shape=_out_sds,
)

_tc_final = pl.pallas_call(
    _tc_final_body,
    grid=(GRID,),
    in_specs=[_agg_spec, _degt_spec],
    out_specs=_row_spec,
    out_shape=_out_sds,
)


@jax.jit
def kernel(inputs, edge_index, W0, b0, W1, b1):
    npe = EPAD - E
    i = jnp.arange(npe, dtype=jnp.int32)
    pad_src = i % N
    pad_dst = TRASH + (i % (ACC_ROWS - N))
    edges = jnp.concatenate(
        [edge_index[0], pad_src, edge_index[1], pad_dst])

    deg_part = _deg_kernel(edges).reshape(2, NW, N)
    degt = jnp.transpose(deg_part, (0, 2, 1))
    b0r = jnp.broadcast_to(b0, (8, D))
    b1r = jnp.broadcast_to(b1, (8, D))

    hn0 = _tc_first(inputs, W0, b0r, degt)
    agg0 = _agg_kernel(hn0, edges)
    hn1 = _tc_mid(agg0, degt, inputs, W1, b1r)
    agg1 = _agg_kernel(hn1, edges)
    return _tc_final(agg1, degt)


# speedup vs baseline: 1.1212x; 1.1212x over previous
import functools

import jax
import jax.numpy as jnp
from jax import lax
from jax.experimental import pallas as pl
from jax.experimental.pallas import tpu as pltpu
from jax.experimental.pallas import tpu_sc as plsc

N = 10000
E = 320000
D = 128
K = 128
NW = 32
ECHUNKS = 80
EPT = ECHUNKS * K
EPAD = NW * EPT
EROWS = E // K
TRASH = N
ACC_ROWS = 10112
SROWS = ACC_ROWS // 16
RB = 1000
GRID = N // RB

_mesh = plsc.VectorSubcoreMesh(core_axis_name="c", subcore_axis_name="s")


@functools.partial(
    pl.kernel,
    out_type=jax.ShapeDtypeStruct((2 * NW * N,), jnp.float32),
    mesh=_mesh,
    scratch_types=[
        pltpu.VMEM((ECHUNKS, 2, K), jnp.int32),
        pltpu.VMEM((N,), jnp.float32),
        pltpu.VMEM((N,), jnp.float32),
    ],
    compiler_params=pltpu.CompilerParams(needs_layout_passes=False),
)
def _deg_kernel(edge_hbm, out_hbm, idx_v, dsrc_v, ddst_v):
    w = lax.axis_index("c") * 16 + lax.axis_index("s")
    pltpu.sync_copy(edge_hbm.at[pl.ds(w * ECHUNKS, ECHUNKS)], idx_v)

    zero16 = jnp.zeros((16,), jnp.float32)

    def zbody(i, _):
        dsrc_v[pl.ds(i * 16, 16)] = zero16
        ddst_v[pl.ds(i * 16, 16)] = zero16
        return 0

    lax.fori_loop(0, N // 16, zbody, 0)

    ones16 = jnp.ones((16,), jnp.float32)

    def body(k, _):
        @pl.when(w * ECHUNKS + k < EROWS)
        def _():
            for j in range(K // 16):
                s_idx = idx_v[k, 0, pl.ds(j * 16, 16)]
                d_idx = idx_v[k, 1, pl.ds(j * 16, 16)]
                plsc.addupdate_scatter(dsrc_v, [s_idx], ones16)
                plsc.addupdate_scatter(ddst_v, [d_idx], ones16)
        return 0

    lax.fori_loop(0, ECHUNKS, body, 0)

    pltpu.sync_copy(dsrc_v, out_hbm.at[pl.ds(w * N, N)])
    pltpu.sync_copy(ddst_v, out_hbm.at[pl.ds((NW + w) * N, N)])


@functools.partial(
    pl.kernel,
    out_type=jax.ShapeDtypeStruct((2, ACC_ROWS, D), jnp.float32),
    mesh=_mesh,
    scratch_types=[
        pltpu.VMEM((3, 2, K), jnp.int32),
        pltpu.VMEM((K, D), jnp.float32),
        pltpu.VMEM((K, D), jnp.float32),
        pltpu.VMEM((K, D), jnp.float32),
        pltpu.VMEM_SHARED((ACC_ROWS, D), jnp.float32),
        pltpu.SemaphoreType.DMA,
        pltpu.SemaphoreType.DMA,
        pltpu.SemaphoreType.DMA,
        pltpu.SemaphoreType.DMA,
        pltpu.SemaphoreType.DMA,
        pltpu.SemaphoreType.DMA,
    ],
    compiler_params=pltpu.CompilerParams(needs_layout_passes=False),
)
def _agg_kernel(h_hbm, edge_hbm, out_hbm, idx_v, row0_v, row1_v, row2_v,
                acc_sh, g0, g1, g2, s0, s1, s2):
    c = lax.axis_index("c")
    s = lax.axis_index("s")
    w = c * 16 + s
    ebase = w * ECHUNKS

    rows = (row0_v, row1_v, row2_v)
    gsem = (g0, g1, g2)
    ssem = (s0, s1, s2)

    def load_idx(q, b):
        pltpu.sync_copy(edge_hbm.at[ebase + q], idx_v.at[b])

    def start_gather(b):
        pltpu.async_copy(h_hbm.at[idx_v.at[b, 0]], rows[b], gsem[b])

    def wait_gather(b):
        pltpu.make_async_copy(h_hbm.at[idx_v.at[b, 0]], rows[b],
                              gsem[b]).wait()

    def start_scatter(b):
        pltpu.async_copy(rows[b], acc_sh.at[idx_v.at[b, 1]], ssem[b],
                         add=True)

    def wait_scatter(b):
        pltpu.make_async_copy(rows[b], acc_sh.at[idx_v.at[b, 1]],
                              ssem[b]).wait()

    load_idx(0, 0)
    start_gather(0)
    load_idx(1, 1)
    start_gather(1)

    zero16 = jnp.zeros((16,), jnp.float32)

    def zbody(i, _):
        for j in range(D // 16):
            row2_v[i, pl.ds(j * 16, 16)] = zero16
        return 0

    lax.fori_loop(0, K, zbody, 0)
    zbase = s * SROWS
    for t in range(SROWS // K):
        pltpu.sync_copy(row2_v, acc_sh.at[pl.ds(zbase + t * K, K)])
    pltpu.sync_copy(row2_v.at[pl.ds(0, SROWS % K)],
                    acc_sh.at[pl.ds(zbase + (SROWS // K) * K, SROWS % K)])
    plsc.subcore_barrier()

    def step(q, b, b2, first, last):
        wait_gather(b)
        start_scatter(b)
        if not last:
            if not first:
                wait_scatter(b2)
            load_idx(q + 2, b2)
            start_gather(b2)

    step(0, 0, 2, True, False)
    step(1, 1, 0, False, False)

    def body(m, _):
        q = 3 * m + 2
        step(q, 2, 1, False, False)
        step(q + 1, 0, 2, False, False)
        step(q + 2, 1, 0, False, False)
        return 0

    lax.fori_loop(0, 25, body, 0)
    step(77, 2, 1, False, False)
    step(78, 0, 2, False, True)
    step(79, 1, 0, False, True)
    wait_scatter(2)
    wait_scatter(0)
    wait_scatter(1)

    plsc.subcore_barrier()
    pltpu.sync_copy(acc_sh.at[pl.ds(s * SROWS, SROWS)],
                    out_hbm.at[c, pl.ds(s * SROWS, SROWS)])


def _rsq(deg_block):
    d = jnp.sum(deg_block, axis=1, keepdims=True)
    return lax.rsqrt(jnp.clip(d, 1.0, None))


def _mm(x, w_ref, b_ref):
    return lax.dot_general(
        x, w_ref[...], (((1,), (1,)), ((), ())),
        preferred_element_type=jnp.float32,
        precision=lax.Precision.HIGHEST,
    ) + b_ref[0:1, :]


def _tc_first_body(x_ref, w_ref, b_ref, degt_ref, o_ref):
    srs = _rsq(degt_ref[0])
    o_ref[...] = _mm(x_ref[...], w_ref, b_ref) * srs


def _tc_mid_body(agg_ref, degt_ref, x_ref, w_ref, b_ref, o_ref):
    irs = _rsq(degt_ref[1])
    srs = _rsq(degt_ref[0])
    p = agg_ref[0] + agg_ref[1]
    x1 = jnp.maximum(p * irs + x_ref[...], 0.0)
    o_ref[...] = _mm(x1, w_ref, b_ref) * srs


def _tc_final_body(agg_ref, degt_ref, o_ref):
    irs = _rsq(degt_ref[1])
    o_ref[...] = (agg_ref[0] + agg_ref[1]) * irs


_row_spec = pl.BlockSpec((RB, D), lambda i: (i, 0))
_w_spec = pl.BlockSpec((D, D), lambda i: (0, 0))
_b_spec = pl.BlockSpec((8, D), lambda i: (0, 0))
_degt_spec = pl.BlockSpec((2, RB, NW), lambda i: (0, i, 0))
_agg_spec = pl.BlockSpec((2, RB, D), lambda i: (0, i, 0))
_out_sds = jax.ShapeDtypeStruct((N, D), jnp.float32)

_tc_first = pl.pallas_call(
    _tc_first_body,
    grid=(GRID,),
    in_specs=[_row_spec, _w_spec, _b_spec, _degt_spec],
    out_specs=_row_spec,
    out_shape=_out_sds,
)

_tc_mid = pl.pallas_call(
    _tc_mid_body,
    grid=(GRID,),
    in_specs=[_agg_spec, _degt_spec, _row_spec, _w_spec, _b_spec],
    out_specs=_row_spec,
    out_shape=_out_sds,
)

_tc_final = pl.pallas_call(
    _tc_final_body,
    grid=(GRID,),
    in_specs=[_agg_spec, _degt_spec],
    out_specs=_row_spec,
    out_shape=_out_sds,
)


@jax.jit
def kernel(inputs, edge_index, W0, b0, W1, b1):
    npe = EPAD - E
    i = jnp.arange(npe, dtype=jnp.int32)
    pad_src = (i % N).reshape(-1, K)
    pad_dst = (TRASH + (i % (ACC_ROWS - N))).reshape(-1, K)
    er = jnp.transpose(edge_index.reshape(2, EROWS, K), (1, 0, 2))
    pad = jnp.stack([pad_src, pad_dst], axis=1)
    edges = jnp.concatenate([er, pad], axis=0)

    deg_part = _deg_kernel(edges).reshape(2, NW, N)
    degt = jnp.transpose(deg_part, (0, 2, 1))
    b0r = jnp.broadcast_to(b0, (8, D))
    b1r = jnp.broadcast_to(b1, (8, D))

    hn0 = _tc_first(inputs, W0, b0r, degt)
    agg0 = _agg_kernel(hn0, edges)
    hn1 = _tc_mid(agg0, degt, inputs, W1, b1r)
    agg1 = _agg_kernel(hn1, edges)
    return _tc_final(agg1, degt)


# speedup vs baseline: 1.1508x; 1.0264x over previous
import functools

import jax
import jax.numpy as jnp
from jax import lax
from jax.experimental import pallas as pl
from jax.experimental.pallas import tpu as pltpu
from jax.experimental.pallas import tpu_sc as plsc

N = 10000
E = 320000
D = 128
K = 128
NW = 32
ECHUNKS = 80
EPT = ECHUNKS * K
EPAD = NW * EPT
EROWS = E // K
TRASH = N
ACC_ROWS = 10112
SROWS = ACC_ROWS // 16
RB = 1000
GRID = N // RB

_mesh = plsc.VectorSubcoreMesh(core_axis_name="c", subcore_axis_name="s")


@functools.partial(
    pl.kernel,
    out_type=jax.ShapeDtypeStruct((2 * NW * N,), jnp.float32),
    mesh=_mesh,
    scratch_types=[
        pltpu.VMEM((ECHUNKS, 2, K), jnp.int32),
        pltpu.VMEM((N,), jnp.float32),
        pltpu.VMEM((N,), jnp.float32),
    ],
    compiler_params=pltpu.CompilerParams(needs_layout_passes=False),
)
def _deg_kernel(edge_hbm, out_hbm, idx_v, dsrc_v, ddst_v):
    w = lax.axis_index("c") * 16 + lax.axis_index("s")
    pltpu.sync_copy(edge_hbm.at[pl.ds(w * ECHUNKS, ECHUNKS)], idx_v)

    zero16 = jnp.zeros((16,), jnp.float32)

    def zbody(i, _):
        dsrc_v[pl.ds(i * 16, 16)] = zero16
        ddst_v[pl.ds(i * 16, 16)] = zero16
        return 0

    lax.fori_loop(0, N // 16, zbody, 0)

    ones16 = jnp.ones((16,), jnp.float32)

    def body(k, _):
        @pl.when(w * ECHUNKS + k < EROWS)
        def _():
            for j in range(K // 16):
                s_idx = idx_v[k, 0, pl.ds(j * 16, 16)]
                d_idx = idx_v[k, 1, pl.ds(j * 16, 16)]
                plsc.addupdate_scatter(dsrc_v, [s_idx], ones16)
                plsc.addupdate_scatter(ddst_v, [d_idx], ones16)
        return 0

    lax.fori_loop(0, ECHUNKS, body, 0)

    pltpu.sync_copy(dsrc_v, out_hbm.at[pl.ds(w * N, N)])
    pltpu.sync_copy(ddst_v, out_hbm.at[pl.ds((NW + w) * N, N)])


@functools.partial(
    pl.kernel,
    out_type=jax.ShapeDtypeStruct((2, ACC_ROWS, D), jnp.float32),
    mesh=_mesh,
    scratch_types=[
        pltpu.VMEM((3, 2, K), jnp.int32),
        pltpu.VMEM((K, D), jnp.float32),
        pltpu.VMEM((K, D), jnp.float32),
        pltpu.VMEM((K, D), jnp.float32),
        pltpu.VMEM_SHARED((ACC_ROWS, D), jnp.float32),
        pltpu.SemaphoreType.DMA,
        pltpu.SemaphoreType.DMA,
        pltpu.SemaphoreType.DMA,
        pltpu.SemaphoreType.DMA,
        pltpu.SemaphoreType.DMA,
        pltpu.SemaphoreType.DMA,
    ],
    compiler_params=pltpu.CompilerParams(needs_layout_passes=False),
)
def _agg_kernel(h_hbm, edge_hbm, out_hbm, idx_v, row0_v, row1_v, row2_v,
                acc_sh, g0, g1, g2, s0, s1, s2):
    c = lax.axis_index("c")
    s = lax.axis_index("s")
    w = c * 16 + s
    ebase = w * ECHUNKS

    rows = (row0_v, row1_v, row2_v)
    gsem = (g0, g1, g2)
    ssem = (s0, s1, s2)

    def load_idx(q, b):
        pltpu.sync_copy(edge_hbm.at[ebase + q], idx_v.at[b])

    def start_gather(b):
        pltpu.async_copy(h_hbm.at[idx_v.at[b, 0]], rows[b], gsem[b])

    def wait_gather(b):
        pltpu.make_async_copy(h_hbm.at[idx_v.at[b, 0]], rows[b],
                              gsem[b]).wait()

    def start_scatter(b):
        pltpu.async_copy(rows[b], acc_sh.at[idx_v.at[b, 1]], ssem[b],
                         add=True)

    def wait_scatter(b):
        pltpu.make_async_copy(rows[b], acc_sh.at[idx_v.at[b, 1]],
                              ssem[b]).wait()

    load_idx(0, 0)
    start_gather(0)
    load_idx(1, 1)
    start_gather(1)

    zero16 = jnp.zeros((16,), jnp.float32)

    def zbody(i, _):
        for j in range(D // 16):
            row2_v[i, pl.ds(j * 16, 16)] = zero16
        return 0

    lax.fori_loop(0, K, zbody, 0)
    zbase = s * SROWS
    for t in range(SROWS // K):
        pltpu.sync_copy(row2_v, acc_sh.at[pl.ds(zbase + t * K, K)])
    pltpu.sync_copy(row2_v.at[pl.ds(0, SROWS % K)],
                    acc_sh.at[pl.ds(zbase + (SROWS // K) * K, SROWS % K)])
    plsc.subcore_barrier()

    def step(q, b, b2, first, last):
        wait_gather(b)
        start_scatter(b)
        if not last:
            if not first:
                wait_scatter(b2)
            load_idx(q + 2, b2)
            start_gather(b2)

    step(0, 0, 2, True, False)
    step(1, 1, 0, False, False)

    def body(m, _):
        q = 3 * m + 2
        step(q, 2, 1, False, False)
        step(q + 1, 0, 2, False, False)
        step(q + 2, 1, 0, False, False)
        return 0

    lax.fori_loop(0, 25, body, 0)
    step(77, 2, 1, False, False)
    step(78, 0, 2, False, True)
    step(79, 1, 0, False, True)
    wait_scatter(2)
    wait_scatter(0)
    wait_scatter(1)

    plsc.subcore_barrier()
    pltpu.sync_copy(acc_sh.at[pl.ds(s * SROWS, SROWS)],
                    out_hbm.at[c, pl.ds(s * SROWS, SROWS)])


def _rsq(deg_block):
    d = jnp.sum(deg_block, axis=1, keepdims=True)
    return lax.rsqrt(jnp.clip(d, 1.0, None))


def _mm(x, w_ref, b_ref):
    return lax.dot_general(
        x, w_ref[...], (((1,), (1,)), ((), ())),
        preferred_element_type=jnp.float32,
        precision=lax.Precision.DEFAULT,
    ) + b_ref[0:1, :]


def _tc_first_body(x_ref, w_ref, b_ref, degt_ref, o_ref):
    srs = _rsq(degt_ref[0])
    o_ref[...] = _mm(x_ref[...], w_ref, b_ref) * srs


def _tc_mid_body(agg_ref, degt_ref, x_ref, w_ref, b_ref, o_ref):
    irs = _rsq(degt_ref[1])
    srs = _rsq(degt_ref[0])
    p = agg_ref[0] + agg_ref[1]
    x1 = jnp.maximum(p * irs + x_ref[...], 0.0)
    o_ref[...] = _mm(x1, w_ref, b_ref) * srs


def _tc_final_body(agg_ref, degt_ref, o_ref):
    irs = _rsq(degt_ref[1])
    o_ref[...] = (agg_ref[0] + agg_ref[1]) * irs


_row_spec = pl.BlockSpec((RB, D), lambda i: (i, 0))
_w_spec = pl.BlockSpec((D, D), lambda i: (0, 0))
_b_spec = pl.BlockSpec((1, D), lambda i: (0, 0))
_degt_spec = pl.BlockSpec((2, RB, NW), lambda i: (0, i, 0))
_agg_spec = pl.BlockSpec((2, RB, D), lambda i: (0, i, 0))
_out_sds = jax.ShapeDtypeStruct((N, D), jnp.float32)

_tc_first = pl.pallas_call(
    _tc_first_body,
    grid=(GRID,),
    in_specs=[_row_spec, _w_spec, _b_spec, _degt_spec],
    out_specs=_row_spec,
    out_shape=_out_sds,
)

_tc_mid = pl.pallas_call(
    _tc_mid_body,
    grid=(GRID,),
    in_specs=[_agg_spec, _degt_spec, _row_spec, _w_spec, _b_spec],
    out_specs=_row_spec,
    out_shape=_out_sds,
)

_tc_final = pl.pallas_call(
    _tc_final_body,
    grid=(GRID,),
    in_specs=[_agg_spec, _degt_spec],
    out_specs=_row_spec,
    out_shape=_out_sds,
)


@jax.jit
def kernel(inputs, edge_index, W0, b0, W1, b1):
    npe = EPAD - E
    i = jnp.arange(npe, dtype=jnp.int32)
    pad_src = (i % N).reshape(-1, K)
    pad_dst = (TRASH + (i % (ACC_ROWS - N))).reshape(-1, K)
    er = jnp.transpose(edge_index.reshape(2, EROWS, K), (1, 0, 2))
    pad = jnp.stack([pad_src, pad_dst], axis=1)
    edges = jnp.concatenate([er, pad], axis=0)

    deg_part = _deg_kernel(edges).reshape(2, NW, N)
    degt = jnp.transpose(deg_part, (0, 2, 1))
    b0r = b0.reshape(1, D)
    b1r = b1.reshape(1, D)

    hn0 = _tc_first(inputs, W0, b0r, degt)
    agg0 = _agg_kernel(hn0, edges)
    hn1 = _tc_mid(agg0, degt, inputs, W1, b1r)
    agg1 = _agg_kernel(hn1, edges)
    return _tc_final(agg1, degt)


# speedup vs baseline: 1.1773x; 1.0230x over previous
import functools

import jax
import jax.numpy as jnp
from jax import lax
from jax.experimental import pallas as pl
from jax.experimental.pallas import tpu as pltpu
from jax.experimental.pallas import tpu_sc as plsc

N = 10000
E = 320000
D = 128
K = 128
NW = 32
ECHUNKS = 80
EPT = ECHUNKS * K
EPAD = NW * EPT
EROWS = E // K
TRASH = N
ACC_ROWS = 10112
SROWS = ACC_ROWS // 16
RB = 2000
GRID = N // RB

_mesh = plsc.VectorSubcoreMesh(core_axis_name="c", subcore_axis_name="s")


@functools.partial(
    pl.kernel,
    out_type=jax.ShapeDtypeStruct((2 * NW * N,), jnp.float32),
    mesh=_mesh,
    scratch_types=[
        pltpu.VMEM((ECHUNKS, 2, K), jnp.int32),
        pltpu.VMEM((N,), jnp.float32),
        pltpu.VMEM((N,), jnp.float32),
    ],
    compiler_params=pltpu.CompilerParams(needs_layout_passes=False),
)
def _deg_kernel(edge_hbm, out_hbm, idx_v, dsrc_v, ddst_v):
    w = lax.axis_index("c") * 16 + lax.axis_index("s")
    pltpu.sync_copy(edge_hbm.at[pl.ds(w * ECHUNKS, ECHUNKS)], idx_v)

    zero16 = jnp.zeros((16,), jnp.float32)

    def zbody(i, _):
        dsrc_v[pl.ds(i * 16, 16)] = zero16
        ddst_v[pl.ds(i * 16, 16)] = zero16
        return 0

    lax.fori_loop(0, N // 16, zbody, 0)

    ones16 = jnp.ones((16,), jnp.float32)

    def body(k, _):
        @pl.when(w * ECHUNKS + k < EROWS)
        def _():
            for j in range(K // 16):
                s_idx = idx_v[k, 0, pl.ds(j * 16, 16)]
                d_idx = idx_v[k, 1, pl.ds(j * 16, 16)]
                plsc.addupdate_scatter(dsrc_v, [s_idx], ones16)
                plsc.addupdate_scatter(ddst_v, [d_idx], ones16)
        return 0

    lax.fori_loop(0, ECHUNKS, body, 0)

    pltpu.sync_copy(dsrc_v, out_hbm.at[pl.ds(w * N, N)])
    pltpu.sync_copy(ddst_v, out_hbm.at[pl.ds((NW + w) * N, N)])


@functools.partial(
    pl.kernel,
    out_type=jax.ShapeDtypeStruct((2, ACC_ROWS, D), jnp.float32),
    mesh=_mesh,
    scratch_types=[
        pltpu.VMEM((3, 2, K), jnp.int32),
        pltpu.VMEM((K, D), jnp.float32),
        pltpu.VMEM((K, D), jnp.float32),
        pltpu.VMEM((K, D), jnp.float32),
        pltpu.VMEM_SHARED((ACC_ROWS, D), jnp.float32),
        pltpu.SemaphoreType.DMA,
        pltpu.SemaphoreType.DMA,
        pltpu.SemaphoreType.DMA,
        pltpu.SemaphoreType.DMA,
        pltpu.SemaphoreType.DMA,
        pltpu.SemaphoreType.DMA,
    ],
    compiler_params=pltpu.CompilerParams(needs_layout_passes=False),
)
def _agg_kernel(h_hbm, edge_hbm, out_hbm, idx_v, row0_v, row1_v, row2_v,
                acc_sh, g0, g1, g2, s0, s1, s2):
    c = lax.axis_index("c")
    s = lax.axis_index("s")
    w = c * 16 + s
    ebase = w * ECHUNKS

    rows = (row0_v, row1_v, row2_v)
    gsem = (g0, g1, g2)
    ssem = (s0, s1, s2)

    def load_idx(q, b):
        pltpu.sync_copy(edge_hbm.at[ebase + q], idx_v.at[b])

    def start_gather(b):
        pltpu.async_copy(h_hbm.at[idx_v.at[b, 0]], rows[b], gsem[b])

    def wait_gather(b):
        pltpu.make_async_copy(h_hbm.at[idx_v.at[b, 0]], rows[b],
                              gsem[b]).wait()

    def start_scatter(b):
        pltpu.async_copy(rows[b], acc_sh.at[idx_v.at[b, 1]], ssem[b],
                         add=True)

    def wait_scatter(b):
        pltpu.make_async_copy(rows[b], acc_sh.at[idx_v.at[b, 1]],
                              ssem[b]).wait()

    load_idx(0, 0)
    start_gather(0)
    load_idx(1, 1)
    start_gather(1)

    zero16 = jnp.zeros((16,), jnp.float32)

    def zbody(i, _):
        for j in range(D // 16):
            row2_v[i, pl.ds(j * 16, 16)] = zero16
        return 0

    lax.fori_loop(0, K, zbody, 0)
    zbase = s * SROWS
    for t in range(SROWS // K):
        pltpu.sync_copy(row2_v, acc_sh.at[pl.ds(zbase + t * K, K)])
    pltpu.sync_copy(row2_v.at[pl.ds(0, SROWS % K)],
                    acc_sh.at[pl.ds(zbase + (SROWS // K) * K, SROWS % K)])
    plsc.subcore_barrier()

    def step(q, b, b2, first, last):
        wait_gather(b)
        start_scatter(b)
        if not last:
            if not first:
                wait_scatter(b2)
            load_idx(q + 2, b2)
            start_gather(b2)

    step(0, 0, 2, True, False)
    step(1, 1, 0, False, False)

    def body(m, _):
        q = 3 * m + 2
        step(q, 2, 1, False, False)
        step(q + 1, 0, 2, False, False)
        step(q + 2, 1, 0, False, False)
        return 0

    lax.fori_loop(0, 25, body, 0)
    step(77, 2, 1, False, False)
    step(78, 0, 2, False, True)
    step(79, 1, 0, False, True)
    wait_scatter(2)
    wait_scatter(0)
    wait_scatter(1)

    plsc.subcore_barrier()
    pltpu.sync_copy(acc_sh.at[pl.ds(s * SROWS, SROWS)],
                    out_hbm.at[c, pl.ds(s * SROWS, SROWS)])


def _rsq(deg_block):
    d = jnp.sum(deg_block, axis=1, keepdims=True)
    return lax.rsqrt(jnp.clip(d, 1.0, None))


def _mm(x, w_ref, b_ref):
    return lax.dot_general(
        x, w_ref[...], (((1,), (1,)), ((), ())),
        preferred_element_type=jnp.float32,
        precision=lax.Precision.DEFAULT,
    ) + b_ref[0:1, :]


def _tc_first_body(x_ref, w_ref, b_ref, degt_ref, o_ref):
    srs = _rsq(degt_ref[0])
    o_ref[...] = _mm(x_ref[...], w_ref, b_ref) * srs


def _tc_mid_body(agg_ref, degt_ref, x_ref, w_ref, b_ref, o_ref):
    irs = _rsq(degt_ref[1])
    srs = _rsq(degt_ref[0])
    p = agg_ref[0] + agg_ref[1]
    x1 = jnp.maximum(p * irs + x_ref[...], 0.0)
    o_ref[...] = _mm(x1, w_ref, b_ref) * srs


def _tc_final_body(agg_ref, degt_ref, o_ref):
    irs = _rsq(degt_ref[1])
    o_ref[...] = (agg_ref[0] + agg_ref[1]) * irs


_row_spec = pl.BlockSpec((RB, D), lambda i: (i, 0))
_w_spec = pl.BlockSpec((D, D), lambda i: (0, 0))
_b_spec = pl.BlockSpec((1, D), lambda i: (0, 0))
_degt_spec = pl.BlockSpec((2, RB, NW), lambda i: (0, i, 0))
_agg_spec = pl.BlockSpec((2, RB, D), lambda i: (0, i, 0))
_out_sds = jax.ShapeDtypeStruct((N, D), jnp.float32)

_tc_first = pl.pallas_call(
    _tc_first_body,
    grid=(GRID,),
    in_specs=[_row_spec, _w_spec, _b_spec, _degt_spec],
    out_specs=_row_spec,
    out_shape=_out_sds,
)

_tc_mid = pl.pallas_call(
    _tc_mid_body,
    grid=(GRID,),
    in_specs=[_agg_spec, _degt_spec, _row_spec, _w_spec, _b_spec],
    out_specs=_row_spec,
    out_shape=_out_sds,
)

_tc_final = pl.pallas_call(
    _tc_final_body,
    grid=(GRID,),
    in_specs=[_agg_spec, _degt_spec],
    out_specs=_row_spec,
    out_shape=_out_sds,
)


@jax.jit
def kernel(inputs, edge_index, W0, b0, W1, b1):
    npe = EPAD - E
    i = jnp.arange(npe, dtype=jnp.int32)
    pad_src = (i % N).reshape(-1, K)
    pad_dst = (TRASH + (i % (ACC_ROWS - N))).reshape(-1, K)
    er = jnp.transpose(edge_index.reshape(2, EROWS, K), (1, 0, 2))
    pad = jnp.stack([pad_src, pad_dst], axis=1)
    edges = jnp.concatenate([er, pad], axis=0)

    deg_part = _deg_kernel(edges).reshape(2, NW, N)
    degt = jnp.transpose(deg_part, (0, 2, 1))
    b0r = b0.reshape(1, D)
    b1r = b1.reshape(1, D)

    hn0 = _tc_first(inputs, W0, b0r, degt)
    agg0 = _agg_kernel(hn0, edges)
    hn1 = _tc_mid(agg0, degt, inputs, W1, b1r)
    agg1 = _agg_kernel(hn1, edges)
    return _tc_final(agg1, degt)
